# Initial kernel scaffold; baseline (speedup 1.0000x reference)
import jax
import jax.numpy as jnp
from jax.experimental import pallas as pl


def kernel(x, edge_index, W1_l, b1, W1_r, W2_l, b2, W2_r):
    raise NotImplementedError("write your pallas kernel here")


# speedup vs baseline: 8.4422x; 8.4422x over previous
import functools

import jax
import jax.numpy as jnp
from jax import lax
from jax.experimental import pallas as pl
from jax.experimental.pallas import tpu as pltpu
from jax.experimental.pallas import tpu_sc as plsc

N_NODES = 10000
N_EDGES = 320000
D = 128

NC = 2
NS = 16
NW = NC * NS

CHUNK = 128
K = 8
KB = 2
EPW = 10240
E_PAD = EPW * NW
N_PAD = N_NODES + 112
ZPS = N_PAD // NS
STEPS = EPW // (CHUNK * K)

_MESH = plsc.VectorSubcoreMesh(core_axis_name="c", subcore_axis_name="s")


def _sc_count(dst_rows, zeros_r, ones_c):
  out_types = jax.ShapeDtypeStruct((NC, N_PAD, D), jnp.float32)
  scratch = [
      pltpu.VMEM((K, CHUNK), jnp.int32),
      pltpu.VMEM((CHUNK, D), jnp.float32),
      pltpu.VMEM_SHARED((N_PAD, D), jnp.float32),
      pltpu.SemaphoreType.DMA,
  ]

  def body(dstR, zc_hbm, ones_hbm, cnt_out, dst_v, ones_v, cnt, ssem):
    cid = lax.axis_index("c")
    sid = lax.axis_index("s")
    w = cid * NS + sid
    rows = pl.ds(sid * ZPS, ZPS)

    pltpu.sync_copy(zc_hbm.at[rows], cnt.at[rows])
    pltpu.sync_copy(ones_hbm, ones_v)
    plsc.subcore_barrier()

    def step(g, carry):
      base = w * (EPW // CHUNK) + g * K
      pltpu.sync_copy(dstR.at[pl.ds(base, K)], dst_v)
      puts = [
          pltpu.async_copy(ones_v, cnt.at[dst_v.at[b]], ssem, add=True)
          for b in range(K)
      ]
      for d in puts:
        d.wait()
      return carry

    lax.fori_loop(0, STEPS, step, 0)
    plsc.subcore_barrier()
    pltpu.sync_copy(cnt.at[rows], cnt_out.at[cid, rows])

  fn = pl.kernel(body, out_type=out_types, mesh=_MESH, scratch_types=scratch)
  return fn(dst_rows, zeros_r, ones_c)


def _sc_aggregate(x, src_rows, dst_rows, zeros_r):
  out_types = jax.ShapeDtypeStruct((NC, N_PAD, D), jnp.float32)
  scratch = [
      pltpu.VMEM((K, CHUNK), jnp.int32),
      pltpu.VMEM((K, CHUNK), jnp.int32),
      pltpu.VMEM((KB, CHUNK, D), jnp.float32),
      pltpu.VMEM_SHARED((N_PAD, D), jnp.float32),
      pltpu.SemaphoreType.DMA,
      pltpu.SemaphoreType.DMA,
  ]

  def body(x_hbm, srcR, dstR, zr_hbm, out, src_v, dst_v, rows_v, acc,
           gsem, ssem):
    cid = lax.axis_index("c")
    sid = lax.axis_index("s")
    w = cid * NS + sid
    rows = pl.ds(sid * ZPS, ZPS)

    pltpu.sync_copy(zr_hbm.at[rows], acc.at[rows])
    plsc.subcore_barrier()

    def step(g, carry):
      base = w * (EPW // CHUNK) + g * K
      pltpu.sync_copy(srcR.at[pl.ds(base, K)], src_v)
      pltpu.sync_copy(dstR.at[pl.ds(base, K)], dst_v)
      for half in range(K // KB):
        gets = [
            pltpu.async_copy(
                x_hbm.at[src_v.at[half * KB + b]], rows_v.at[b], gsem)
            for b in range(KB)
        ]
        for d in gets:
          d.wait()
        puts = [
            pltpu.async_copy(rows_v.at[b], acc.at[dst_v.at[half * KB + b]],
                             ssem, add=True)
            for b in range(KB)
        ]
        for d in puts:
          d.wait()
      return carry

    lax.fori_loop(0, STEPS, step, 0)
    plsc.subcore_barrier()

    pltpu.sync_copy(acc.at[rows], out.at[cid, rows])

  fn = pl.kernel(body, out_type=out_types, mesh=_MESH, scratch_types=scratch)
  return fn(x, src_rows, dst_rows, zeros_r)


def _tc_layer(p, c, xin, wl, wr, bias, relu):
  BN = 2000
  grid = (N_NODES // BN,)

  def body(p0_r, p1_r, c0_r, c1_r, x_r, wl_r, wr_r, b_r, o_r):
    cntv = jnp.maximum(c0_r[0, :, :1] + c1_r[0, :, :1], 1.0)
    mean = (p0_r[0] + p1_r[0]) / cntv
    acc = lax.dot_general(mean, wl_r[...], (((1,), (1,)), ((), ())),
                          preferred_element_type=jnp.float32)
    acc = acc + lax.dot_general(x_r[...], wr_r[...], (((1,), (1,)), ((), ())),
                                preferred_element_type=jnp.float32)
    acc = acc + b_r[0:1, :]
    o_r[...] = jnp.maximum(acc, 0.0) if relu else acc

  p0s = pl.BlockSpec((1, BN, D), lambda i: (0, i, 0))
  p1s = pl.BlockSpec((1, BN, D), lambda i: (1, i, 0))
  c0s = pl.BlockSpec((1, BN, D), lambda i: (0, i, 0))
  c1s = pl.BlockSpec((1, BN, D), lambda i: (1, i, 0))
  big = pl.BlockSpec((BN, D), lambda i: (i, 0))
  wspec = pl.BlockSpec((D, D), lambda i: (0, 0))
  bspec = pl.BlockSpec((8, D), lambda i: (0, 0))

  return pl.pallas_call(
      body,
      grid=grid,
      in_specs=[p0s, p1s, c0s, c1s, big, wspec, wspec, bspec],
      out_specs=big,
      out_shape=jax.ShapeDtypeStruct((N_NODES, D), jnp.float32),
  )(p, p, c, c, xin, wl, wr, bias)


@jax.jit
def kernel(x, edge_index, W1_l, b1, W1_r, W2_l, b2, W2_r):
  src = edge_index[0].astype(jnp.int32)
  dst = edge_index[1].astype(jnp.int32)

  pad_n = E_PAD - N_EDGES
  pad_ar = jnp.arange(pad_n, dtype=jnp.int32)
  pad_src = (pad_ar * 131) % N_NODES
  pad_dst = N_NODES + (pad_ar % 112)
  src_rows = jnp.concatenate([src, pad_src]).reshape(E_PAD // CHUNK, CHUNK)
  dst_rows = jnp.concatenate([dst, pad_dst]).reshape(E_PAD // CHUNK, CHUNK)

  zeros_r = jnp.zeros((N_PAD, D), jnp.float32)
  ones_c = jnp.ones((CHUNK, D), jnp.float32)

  b1r = jnp.broadcast_to(b1, (8, D))
  b2r = jnp.broadcast_to(b2, (8, D))

  c = _sc_count(dst_rows, zeros_r, ones_c)
  p = _sc_aggregate(x, src_rows, dst_rows, zeros_r)
  h = _tc_layer(p, c, x, W1_l, W1_r, b1r, relu=True)

  q = _sc_aggregate(h, src_rows, dst_rows, zeros_r)
  out = _tc_layer(q, c, h, W2_l, W2_r, b2r, relu=False)
  return out


# speedup vs baseline: 10.3680x; 1.2281x over previous
import functools

import jax
import jax.numpy as jnp
from jax import lax
from jax.experimental import pallas as pl
from jax.experimental.pallas import tpu as pltpu
from jax.experimental.pallas import tpu_sc as plsc

N_NODES = 10000
N_EDGES = 320000
D = 128

NC = 2
NS = 16
NW = NC * NS

CHUNK = 128
K = 8
KB = 2
EPW = 10240
E_PAD = EPW * NW
N_PAD = N_NODES + 112
ZPS = N_PAD // NS
STEPS = EPW // (CHUNK * K)

_MESH = plsc.VectorSubcoreMesh(core_axis_name="c", subcore_axis_name="s")


def _sc_count(dst_rows, zeros_r, ones_c):
  out_types = jax.ShapeDtypeStruct((NC, N_PAD, D), jnp.float32)
  scratch = [
      pltpu.VMEM((K, CHUNK), jnp.int32),
      pltpu.VMEM((CHUNK, D), jnp.float32),
      pltpu.VMEM_SHARED((N_PAD, D), jnp.float32),
      pltpu.SemaphoreType.DMA,
  ]

  def body(dstR, zc_hbm, ones_hbm, cnt_out, dst_v, ones_v, cnt, ssem):
    cid = lax.axis_index("c")
    sid = lax.axis_index("s")
    w = cid * NS + sid
    rows = pl.ds(sid * ZPS, ZPS)

    pltpu.sync_copy(zc_hbm.at[rows], cnt.at[rows])
    pltpu.sync_copy(ones_hbm, ones_v)
    plsc.subcore_barrier()

    def step(g, carry):
      base = w * (EPW // CHUNK) + g * K
      pltpu.sync_copy(dstR.at[pl.ds(base, K)], dst_v)
      puts = [
          pltpu.async_copy(ones_v, cnt.at[dst_v.at[b]], ssem, add=True)
          for b in range(K)
      ]
      for d in puts:
        d.wait()
      return carry

    lax.fori_loop(0, STEPS, step, 0)
    plsc.subcore_barrier()
    pltpu.sync_copy(cnt.at[rows], cnt_out.at[cid, rows])

  fn = pl.kernel(body, out_type=out_types, mesh=_MESH, scratch_types=scratch)
  return fn(dst_rows, zeros_r, ones_c)


def _sc_aggregate(x, src_rows, dst_rows, zeros_r):
  out_types = jax.ShapeDtypeStruct((NC, N_PAD, D), jnp.float32)
  P = (EPW // CHUNK) // 2
  scratch = [
      pltpu.VMEM((P, CHUNK), jnp.int32),
      pltpu.VMEM((P, CHUNK), jnp.int32),
      pltpu.VMEM((KB, CHUNK, D), jnp.float32),
      pltpu.VMEM_SHARED((N_PAD, D), jnp.float32),
      pltpu.SemaphoreType.DMA,
      pltpu.SemaphoreType.DMA,
  ]

  def body(x_hbm, srcR, dstR, zr_hbm, out, src_v, dst_v, rows_v, acc,
           gsem, ssem):
    cid = lax.axis_index("c")
    sid = lax.axis_index("s")
    w = cid * NS + sid
    rows = pl.ds(sid * ZPS, ZPS)

    pltpu.sync_copy(zr_hbm.at[rows], acc.at[rows])
    plsc.subcore_barrier()

    def gather(c, b):
      return pltpu.async_copy(x_hbm.at[src_v.at[c]], rows_v.at[b], gsem)

    def wait_gather(b):
      pltpu.make_async_copy(x_hbm.at[src_v.at[0]], rows_v.at[b], gsem).wait()

    def scatter(c, b):
      return pltpu.async_copy(rows_v.at[b], acc.at[dst_v.at[c]], ssem,
                              add=True)

    for phase in range(2):
      base = w * (EPW // CHUNK) + phase * P
      pltpu.sync_copy(srcR.at[pl.ds(base, P)], src_v)
      pltpu.sync_copy(dstR.at[pl.ds(base, P)], dst_v)
      gather(0, 0)
      gather(1, 1)

      def ring(g, carry):
        c0 = 2 * g
        c1 = 2 * g + 1
        n0 = lax.rem(c0 + 2, P)
        n1 = lax.rem(c1 + 2, P)
        wait_gather(0)
        s0 = scatter(c0, 0)
        wait_gather(1)
        s1 = scatter(c1, 1)
        s0.wait()
        gather(n0, 0)
        s1.wait()
        gather(n1, 1)
        return carry

      lax.fori_loop(0, P // 2, ring, 0)
      wait_gather(0)
      wait_gather(1)

    plsc.subcore_barrier()

    pltpu.sync_copy(acc.at[rows], out.at[cid, rows])

  fn = pl.kernel(body, out_type=out_types, mesh=_MESH, scratch_types=scratch)
  return fn(x, src_rows, dst_rows, zeros_r)


def _tc_layer(p, c, xin, wl, wr, bias, relu):
  BN = 2000
  grid = (N_NODES // BN,)

  def body(p0_r, p1_r, c0_r, c1_r, x_r, wl_r, wr_r, b_r, o_r):
    cntv = jnp.maximum(c0_r[0, :, :1] + c1_r[0, :, :1], 1.0)
    mean = (p0_r[0] + p1_r[0]) / cntv
    acc = lax.dot_general(mean, wl_r[...], (((1,), (1,)), ((), ())),
                          preferred_element_type=jnp.float32)
    acc = acc + lax.dot_general(x_r[...], wr_r[...], (((1,), (1,)), ((), ())),
                                preferred_element_type=jnp.float32)
    acc = acc + b_r[0:1, :]
    o_r[...] = jnp.maximum(acc, 0.0) if relu else acc

  p0s = pl.BlockSpec((1, BN, D), lambda i: (0, i, 0))
  p1s = pl.BlockSpec((1, BN, D), lambda i: (1, i, 0))
  c0s = pl.BlockSpec((1, BN, D), lambda i: (0, i, 0))
  c1s = pl.BlockSpec((1, BN, D), lambda i: (1, i, 0))
  big = pl.BlockSpec((BN, D), lambda i: (i, 0))
  wspec = pl.BlockSpec((D, D), lambda i: (0, 0))
  bspec = pl.BlockSpec((8, D), lambda i: (0, 0))

  return pl.pallas_call(
      body,
      grid=grid,
      in_specs=[p0s, p1s, c0s, c1s, big, wspec, wspec, bspec],
      out_specs=big,
      out_shape=jax.ShapeDtypeStruct((N_NODES, D), jnp.float32),
  )(p, p, c, c, xin, wl, wr, bias)


@jax.jit
def kernel(x, edge_index, W1_l, b1, W1_r, W2_l, b2, W2_r):
  src = edge_index[0].astype(jnp.int32)
  dst = edge_index[1].astype(jnp.int32)

  pad_n = E_PAD - N_EDGES
  pad_ar = jnp.arange(pad_n, dtype=jnp.int32)
  pad_src = (pad_ar * 131) % N_NODES
  pad_dst = N_NODES + (pad_ar % 112)
  src_rows = jnp.concatenate([src, pad_src]).reshape(E_PAD // CHUNK, CHUNK)
  dst_rows = jnp.concatenate([dst, pad_dst]).reshape(E_PAD // CHUNK, CHUNK)

  zeros_r = jnp.zeros((N_PAD, D), jnp.float32)
  ones_c = jnp.ones((CHUNK, D), jnp.float32)

  b1r = jnp.broadcast_to(b1, (8, D))
  b2r = jnp.broadcast_to(b2, (8, D))

  c = _sc_count(dst_rows, zeros_r, ones_c)
  p = _sc_aggregate(x, src_rows, dst_rows, zeros_r)
  h = _tc_layer(p, c, x, W1_l, W1_r, b1r, relu=True)

  q = _sc_aggregate(h, src_rows, dst_rows, zeros_r)
  out = _tc_layer(q, c, h, W2_l, W2_r, b2r, relu=False)
  return out


# speedup vs baseline: 10.4570x; 1.0086x over previous
import functools

import jax
import jax.numpy as jnp
from jax import lax
from jax.experimental import pallas as pl
from jax.experimental.pallas import tpu as pltpu
from jax.experimental.pallas import tpu_sc as plsc

N_NODES = 10000
N_EDGES = 320000
D = 128

NC = 2
NS = 16
NW = NC * NS

CHUNK = 128
K = 8
KB = 2
EPW = 10240
E_PAD = EPW * NW
N_PAD = N_NODES + 112
ZPS = N_PAD // NS
STEPS = EPW // (CHUNK * K)

_MESH = plsc.VectorSubcoreMesh(core_axis_name="c", subcore_axis_name="s")


def _sc_count(dst_rows, zeros_r, ones_c):
  NCH = EPW // CHUNK
  out_types = jax.ShapeDtypeStruct((NC, N_PAD, D), jnp.float32)
  scratch = [
      pltpu.VMEM((NCH, CHUNK), jnp.int32),
      pltpu.VMEM((CHUNK, D), jnp.float32),
      pltpu.VMEM_SHARED((N_PAD, D), jnp.float32),
      pltpu.SemaphoreType.DMA,
  ]

  def body(dstR, zc_hbm, ones_hbm, cnt_out, dst_v, ones_v, cnt, ssem):
    cid = lax.axis_index("c")
    sid = lax.axis_index("s")
    w = cid * NS + sid
    rows = pl.ds(sid * ZPS, ZPS)

    pltpu.sync_copy(zc_hbm.at[rows], cnt.at[rows])
    pltpu.sync_copy(ones_hbm, ones_v)
    pltpu.sync_copy(dstR.at[pl.ds(w * NCH, NCH)], dst_v)
    plsc.subcore_barrier()

    def step(g, carry):
      puts = [
          pltpu.async_copy(ones_v, cnt.at[dst_v.at[g * K + b]], ssem,
                           add=True)
          for b in range(K)
      ]
      for d in puts:
        d.wait()
      return carry

    lax.fori_loop(0, NCH // K, step, 0)
    plsc.subcore_barrier()
    pltpu.sync_copy(cnt.at[rows], cnt_out.at[cid, rows])

  fn = pl.kernel(body, out_type=out_types, mesh=_MESH, scratch_types=scratch)
  return fn(dst_rows, zeros_r, ones_c)


def _sc_aggregate(x, src_rows, dst_rows, zeros_r):
  out_types = jax.ShapeDtypeStruct((NC, N_PAD, D), jnp.float32)
  P = (EPW // CHUNK) // 2
  scratch = [
      pltpu.VMEM((P, CHUNK), jnp.int32),
      pltpu.VMEM((P, CHUNK), jnp.int32),
      pltpu.VMEM((KB, CHUNK, D), jnp.float32),
      pltpu.VMEM_SHARED((N_PAD, D), jnp.float32),
      pltpu.SemaphoreType.DMA,
      pltpu.SemaphoreType.DMA,
  ]

  def body(x_hbm, srcR, dstR, zr_hbm, out, src_v, dst_v, rows_v, acc,
           gsem, ssem):
    cid = lax.axis_index("c")
    sid = lax.axis_index("s")
    w = cid * NS + sid
    rows = pl.ds(sid * ZPS, ZPS)

    pltpu.sync_copy(zr_hbm.at[rows], acc.at[rows])
    plsc.subcore_barrier()

    def gather(c, b):
      return pltpu.async_copy(x_hbm.at[src_v.at[c]], rows_v.at[b], gsem)

    def wait_gather(b):
      pltpu.make_async_copy(x_hbm.at[src_v.at[0]], rows_v.at[b], gsem).wait()

    def scatter(c, b):
      return pltpu.async_copy(rows_v.at[b], acc.at[dst_v.at[c]], ssem,
                              add=True)

    for phase in range(2):
      base = w * (EPW // CHUNK) + phase * P
      pltpu.sync_copy(srcR.at[pl.ds(base, P)], src_v)
      pltpu.sync_copy(dstR.at[pl.ds(base, P)], dst_v)
      gather(0, 0)
      gather(1, 1)

      def ring(g, carry):
        c0 = 2 * g
        c1 = 2 * g + 1
        n0 = lax.rem(c0 + 2, P)
        n1 = lax.rem(c1 + 2, P)
        wait_gather(0)
        s0 = scatter(c0, 0)
        wait_gather(1)
        s1 = scatter(c1, 1)
        s0.wait()
        gather(n0, 0)
        s1.wait()
        gather(n1, 1)
        return carry

      lax.fori_loop(0, P // 2, ring, 0)
      wait_gather(0)
      wait_gather(1)

    plsc.subcore_barrier()

    pltpu.sync_copy(acc.at[rows], out.at[cid, rows])

  fn = pl.kernel(body, out_type=out_types, mesh=_MESH, scratch_types=scratch)
  return fn(x, src_rows, dst_rows, zeros_r)


def _tc_layer(p, c, xin, wl, wr, bias, relu):
  BN = 2000
  grid = (N_NODES // BN,)

  def body(p0_r, p1_r, c0_r, c1_r, x_r, wl_r, wr_r, b_r, o_r):
    cntv = jnp.maximum(c0_r[0, :, :1] + c1_r[0, :, :1], 1.0)
    mean = (p0_r[0] + p1_r[0]) / cntv
    acc = lax.dot_general(mean, wl_r[...], (((1,), (1,)), ((), ())),
                          preferred_element_type=jnp.float32)
    acc = acc + lax.dot_general(x_r[...], wr_r[...], (((1,), (1,)), ((), ())),
                                preferred_element_type=jnp.float32)
    acc = acc + b_r[0:1, :]
    o_r[...] = jnp.maximum(acc, 0.0) if relu else acc

  p0s = pl.BlockSpec((1, BN, D), lambda i: (0, i, 0))
  p1s = pl.BlockSpec((1, BN, D), lambda i: (1, i, 0))
  c0s = pl.BlockSpec((1, BN, D), lambda i: (0, i, 0))
  c1s = pl.BlockSpec((1, BN, D), lambda i: (1, i, 0))
  big = pl.BlockSpec((BN, D), lambda i: (i, 0))
  wspec = pl.BlockSpec((D, D), lambda i: (0, 0))
  bspec = pl.BlockSpec((8, D), lambda i: (0, 0))

  return pl.pallas_call(
      body,
      grid=grid,
      in_specs=[p0s, p1s, c0s, c1s, big, wspec, wspec, bspec],
      out_specs=big,
      out_shape=jax.ShapeDtypeStruct((N_NODES, D), jnp.float32),
  )(p, p, c, c, xin, wl, wr, bias)


@jax.jit
def kernel(x, edge_index, W1_l, b1, W1_r, W2_l, b2, W2_r):
  src = edge_index[0].astype(jnp.int32)
  dst = edge_index[1].astype(jnp.int32)

  pad_n = E_PAD - N_EDGES
  pad_ar = jnp.arange(pad_n, dtype=jnp.int32)
  pad_src = (pad_ar * 131) % N_NODES
  pad_dst = N_NODES + (pad_ar % 112)
  src_rows = jnp.concatenate([src, pad_src]).reshape(E_PAD // CHUNK, CHUNK)
  dst_rows = jnp.concatenate([dst, pad_dst]).reshape(E_PAD // CHUNK, CHUNK)

  zeros_r = jnp.zeros((N_PAD, D), jnp.float32)
  ones_c = jnp.ones((CHUNK, D), jnp.float32)

  b1r = jnp.broadcast_to(b1, (8, D))
  b2r = jnp.broadcast_to(b2, (8, D))

  c = _sc_count(dst_rows, zeros_r, ones_c)
  p = _sc_aggregate(x, src_rows, dst_rows, zeros_r)
  h = _tc_layer(p, c, x, W1_l, W1_r, b1r, relu=True)

  q = _sc_aggregate(h, src_rows, dst_rows, zeros_r)
  out = _tc_layer(q, c, h, W2_l, W2_r, b2r, relu=False)
  return out


# speedup vs baseline: 10.5827x; 1.0120x over previous
import functools

import jax
import jax.numpy as jnp
from jax import lax
from jax.experimental import pallas as pl
from jax.experimental.pallas import tpu as pltpu
from jax.experimental.pallas import tpu_sc as plsc

N_NODES = 10000
N_EDGES = 320000
D = 128

NC = 2
NS = 16
NW = NC * NS

CHUNK = 128
K = 8
KB = 2
EPW = 10240
E_PAD = EPW * NW
N_PAD = N_NODES + 112
ZPS = N_PAD // NS
STEPS = EPW // (CHUNK * K)

_MESH = plsc.VectorSubcoreMesh(core_axis_name="c", subcore_axis_name="s")


def _sc_count(dst_rows, zeros_r, ones_c):
  NCH = EPW // CHUNK
  out_types = jax.ShapeDtypeStruct((NC, N_PAD, D), jnp.float32)
  scratch = [
      pltpu.VMEM((NCH, CHUNK), jnp.int32),
      pltpu.VMEM((CHUNK, D), jnp.float32),
      pltpu.VMEM_SHARED((N_PAD, D), jnp.float32),
      pltpu.SemaphoreType.DMA,
  ]

  def body(dstR, zc_hbm, ones_hbm, cnt_out, dst_v, ones_v, cnt, ssem):
    cid = lax.axis_index("c")
    sid = lax.axis_index("s")
    w = cid * NS + sid
    rows = pl.ds(sid * ZPS, ZPS)

    pltpu.sync_copy(zc_hbm.at[rows], cnt.at[rows])
    pltpu.sync_copy(ones_hbm, ones_v)
    pltpu.sync_copy(dstR.at[pl.ds(w * NCH, NCH)], dst_v)
    plsc.subcore_barrier()

    def step(g, carry):
      puts = [
          pltpu.async_copy(ones_v, cnt.at[dst_v.at[g * K + b]], ssem,
                           add=True)
          for b in range(K)
      ]
      for d in puts:
        d.wait()
      return carry

    lax.fori_loop(0, NCH // K, step, 0)
    plsc.subcore_barrier()
    pltpu.sync_copy(cnt.at[rows], cnt_out.at[cid, rows])

  fn = pl.kernel(body, out_type=out_types, mesh=_MESH, scratch_types=scratch)
  return fn(dst_rows, zeros_r, ones_c)


def _sc_aggregate(x, src_rows, dst_rows, zeros_r):
  out_types = jax.ShapeDtypeStruct((NC, N_PAD, D), jnp.float32)
  P = (EPW // CHUNK) // 2
  scratch = [
      pltpu.VMEM((P, CHUNK), jnp.int32),
      pltpu.VMEM((P, CHUNK), jnp.int32),
      pltpu.VMEM((KB, CHUNK, D), jnp.float32),
      pltpu.VMEM_SHARED((N_PAD, D), jnp.float32),
      pltpu.SemaphoreType.DMA,
      pltpu.SemaphoreType.DMA,
  ]

  def body(x_hbm, srcR, dstR, zr_hbm, out, src_v, dst_v, rows_v, acc,
           gsem, ssem):
    cid = lax.axis_index("c")
    sid = lax.axis_index("s")
    w = cid * NS + sid
    rows = pl.ds(sid * ZPS, ZPS)

    def gather(c, b):
      return pltpu.async_copy(x_hbm.at[src_v.at[c]], rows_v.at[b], gsem)

    def wait_gather(b):
      pltpu.make_async_copy(x_hbm.at[src_v.at[0]], rows_v.at[b], gsem).wait()

    def scatter(c, b):
      return pltpu.async_copy(rows_v.at[b], acc.at[dst_v.at[c]], ssem,
                              add=True)

    for phase in range(2):
      base = w * (EPW // CHUNK) + phase * P
      pltpu.sync_copy(srcR.at[pl.ds(base, P)], src_v)
      pltpu.sync_copy(dstR.at[pl.ds(base, P)], dst_v)
      gather(0, 0)
      gather(1, 1)
      if phase == 0:
        pltpu.sync_copy(zr_hbm.at[rows], acc.at[rows])
        plsc.subcore_barrier()

      def ring(g, carry):
        c0 = 2 * g
        c1 = 2 * g + 1
        n0 = lax.rem(c0 + 2, P)
        n1 = lax.rem(c1 + 2, P)
        wait_gather(0)
        s0 = scatter(c0, 0)
        wait_gather(1)
        s1 = scatter(c1, 1)
        s0.wait()
        gather(n0, 0)
        s1.wait()
        gather(n1, 1)
        return carry

      lax.fori_loop(0, P // 2, ring, 0)
      wait_gather(0)
      wait_gather(1)

    plsc.subcore_barrier()

    pltpu.sync_copy(acc.at[rows], out.at[cid, rows])

  fn = pl.kernel(body, out_type=out_types, mesh=_MESH, scratch_types=scratch)
  return fn(x, src_rows, dst_rows, zeros_r)


def _tc_layer(p, c, xin, wl, wr, bias, relu):
  BN = 2000
  grid = (N_NODES // BN,)

  def body(p0_r, p1_r, c0_r, c1_r, x_r, wl_r, wr_r, b_r, o_r):
    cntv = jnp.maximum(c0_r[0, :, :1] + c1_r[0, :, :1], 1.0)
    mean = (p0_r[0] + p1_r[0]) / cntv
    acc = lax.dot_general(mean, wl_r[...], (((1,), (1,)), ((), ())),
                          preferred_element_type=jnp.float32)
    acc = acc + lax.dot_general(x_r[...], wr_r[...], (((1,), (1,)), ((), ())),
                                preferred_element_type=jnp.float32)
    acc = acc + b_r[0:1, :]
    o_r[...] = jnp.maximum(acc, 0.0) if relu else acc

  p0s = pl.BlockSpec((1, BN, D), lambda i: (0, i, 0))
  p1s = pl.BlockSpec((1, BN, D), lambda i: (1, i, 0))
  c0s = pl.BlockSpec((1, BN, D), lambda i: (0, i, 0))
  c1s = pl.BlockSpec((1, BN, D), lambda i: (1, i, 0))
  big = pl.BlockSpec((BN, D), lambda i: (i, 0))
  wspec = pl.BlockSpec((D, D), lambda i: (0, 0))
  bspec = pl.BlockSpec((8, D), lambda i: (0, 0))

  return pl.pallas_call(
      body,
      grid=grid,
      in_specs=[p0s, p1s, c0s, c1s, big, wspec, wspec, bspec],
      out_specs=big,
      out_shape=jax.ShapeDtypeStruct((N_NODES, D), jnp.float32),
  )(p, p, c, c, xin, wl, wr, bias)


@jax.jit
def kernel(x, edge_index, W1_l, b1, W1_r, W2_l, b2, W2_r):
  src = edge_index[0].astype(jnp.int32)
  dst = edge_index[1].astype(jnp.int32)

  pad_n = E_PAD - N_EDGES
  pad_ar = jnp.arange(pad_n, dtype=jnp.int32)
  pad_src = (pad_ar * 131) % N_NODES
  pad_dst = N_NODES + (pad_ar % 112)
  src_rows = jnp.concatenate([src, pad_src]).reshape(E_PAD // CHUNK, CHUNK)
  dst_rows = jnp.concatenate([dst, pad_dst]).reshape(E_PAD // CHUNK, CHUNK)

  zeros_r = jnp.zeros((N_PAD, D), jnp.float32)
  ones_c = jnp.ones((CHUNK, D), jnp.float32)

  b1r = jnp.broadcast_to(b1, (8, D))
  b2r = jnp.broadcast_to(b2, (8, D))

  c = _sc_count(dst_rows, zeros_r, ones_c)
  p = _sc_aggregate(x, src_rows, dst_rows, zeros_r)
  h = _tc_layer(p, c, x, W1_l, W1_r, b1r, relu=True)

  q = _sc_aggregate(h, src_rows, dst_rows, zeros_r)
  out = _tc_layer(q, c, h, W2_l, W2_r, b2r, relu=False)
  return out


# speedup vs baseline: 10.5858x; 1.0003x over previous
import jax
import jax.numpy as jnp
from jax import lax
from jax.experimental import pallas as pl
from jax.experimental.pallas import tpu as pltpu
from jax.experimental.pallas import tpu_sc as plsc

N_NODES = 10000
N_EDGES = 320000
D = 128

NC = 2
NS = 16

CHUNK = 128
K = 8
KB = 2
EPW = 10240
E_PAD = EPW * NC * NS
N_PAD = N_NODES + 112
ZPS = N_PAD // NS

_MESH = plsc.VectorSubcoreMesh(core_axis_name="c", subcore_axis_name="s")


def _sc_count(dst_rows, zeros_r, ones_c):
  NCH = EPW // CHUNK
  out_types = jax.ShapeDtypeStruct((NC, N_PAD, D), jnp.float32)
  scratch = [
      pltpu.VMEM((NCH, CHUNK), jnp.int32),
      pltpu.VMEM((CHUNK, D), jnp.float32),
      pltpu.VMEM_SHARED((N_PAD, D), jnp.float32),
      pltpu.SemaphoreType.DMA,
  ]

  def body(dstR, zc_hbm, ones_hbm, cnt_out, dst_v, ones_v, cnt, ssem):
    cid = lax.axis_index("c")
    sid = lax.axis_index("s")
    w = cid * NS + sid
    rows = pl.ds(sid * ZPS, ZPS)

    pltpu.sync_copy(zc_hbm.at[rows], cnt.at[rows])
    pltpu.sync_copy(ones_hbm, ones_v)
    pltpu.sync_copy(dstR.at[pl.ds(w * NCH, NCH)], dst_v)
    plsc.subcore_barrier()

    def step(g, carry):
      puts = [
          pltpu.async_copy(ones_v, cnt.at[dst_v.at[g * K + b]], ssem,
                           add=True)
          for b in range(K)
      ]
      for d in puts:
        d.wait()
      return carry

    lax.fori_loop(0, NCH // K, step, 0)
    plsc.subcore_barrier()
    pltpu.sync_copy(cnt.at[rows], cnt_out.at[cid, rows])

  fn = pl.kernel(body, out_type=out_types, mesh=_MESH, scratch_types=scratch)
  return fn(dst_rows, zeros_r, ones_c)


def _sc_aggregate(x, src_rows, dst_rows, zeros_r):
  out_types = jax.ShapeDtypeStruct((NC, N_PAD, D), jnp.float32)
  P = (EPW // CHUNK) // 2
  scratch = [
      pltpu.VMEM((P, CHUNK), jnp.int32),
      pltpu.VMEM((P, CHUNK), jnp.int32),
      pltpu.VMEM((KB, CHUNK, D), jnp.float32),
      pltpu.VMEM_SHARED((N_PAD, D), jnp.float32),
      pltpu.SemaphoreType.DMA,
      pltpu.SemaphoreType.DMA,
  ]

  def body(x_hbm, srcR, dstR, zr_hbm, out, src_v, dst_v, rows_v, acc,
           gsem, ssem):
    cid = lax.axis_index("c")
    sid = lax.axis_index("s")
    w = cid * NS + sid
    rows = pl.ds(sid * ZPS, ZPS)

    def gather(c, b):
      return pltpu.async_copy(x_hbm.at[src_v.at[c]], rows_v.at[b], gsem)

    def wait_gather(b):
      pltpu.make_async_copy(x_hbm.at[src_v.at[0]], rows_v.at[b], gsem).wait()

    def scatter(c, b):
      return pltpu.async_copy(rows_v.at[b], acc.at[dst_v.at[c]], ssem,
                              add=True)

    for phase in range(2):
      base = w * (EPW // CHUNK) + phase * P
      pltpu.sync_copy(srcR.at[pl.ds(base, P)], src_v)
      pltpu.sync_copy(dstR.at[pl.ds(base, P)], dst_v)
      gather(0, 0)
      gather(1, 1)
      if phase == 0:
        pltpu.sync_copy(zr_hbm.at[rows], acc.at[rows])
        plsc.subcore_barrier()

      def ring(g, carry):
        c0 = 2 * g
        c1 = 2 * g + 1
        n0 = lax.rem(c0 + 2, P)
        n1 = lax.rem(c1 + 2, P)
        wait_gather(0)
        s0 = scatter(c0, 0)
        wait_gather(1)
        s1 = scatter(c1, 1)
        s0.wait()
        gather(n0, 0)
        s1.wait()
        gather(n1, 1)
        return carry

      lax.fori_loop(0, P // 2, ring, 0)
      wait_gather(0)
      wait_gather(1)

    plsc.subcore_barrier()

    pltpu.sync_copy(acc.at[rows], out.at[cid, rows])

  fn = pl.kernel(body, out_type=out_types, mesh=_MESH, scratch_types=scratch)
  return fn(x, src_rows, dst_rows, zeros_r)


def _tc_layer(p, c, xin, wl, wr, bias, relu):
  BN = 2000
  grid = (N_NODES // BN,)

  def body(p0_r, p1_r, c0_r, c1_r, x_r, wl_r, wr_r, b_r, o_r):
    cntv = jnp.maximum(c0_r[0, :, :1] + c1_r[0, :, :1], 1.0)
    mean = (p0_r[0] + p1_r[0]) / cntv
    acc = lax.dot_general(mean, wl_r[...], (((1,), (1,)), ((), ())),
                          preferred_element_type=jnp.float32)
    acc = acc + lax.dot_general(x_r[...], wr_r[...], (((1,), (1,)), ((), ())),
                                preferred_element_type=jnp.float32)
    acc = acc + b_r[0:1, :]
    o_r[...] = jnp.maximum(acc, 0.0) if relu else acc

  p0s = pl.BlockSpec((1, BN, D), lambda i: (0, i, 0))
  p1s = pl.BlockSpec((1, BN, D), lambda i: (1, i, 0))
  c0s = pl.BlockSpec((1, BN, D), lambda i: (0, i, 0))
  c1s = pl.BlockSpec((1, BN, D), lambda i: (1, i, 0))
  big = pl.BlockSpec((BN, D), lambda i: (i, 0))
  wspec = pl.BlockSpec((D, D), lambda i: (0, 0))
  bspec = pl.BlockSpec((8, D), lambda i: (0, 0))

  return pl.pallas_call(
      body,
      grid=grid,
      in_specs=[p0s, p1s, c0s, c1s, big, wspec, wspec, bspec],
      out_specs=big,
      out_shape=jax.ShapeDtypeStruct((N_NODES, D), jnp.float32),
  )(p, p, c, c, xin, wl, wr, bias)


@jax.jit
def kernel(x, edge_index, W1_l, b1, W1_r, W2_l, b2, W2_r):
  src = edge_index[0].astype(jnp.int32)
  dst = edge_index[1].astype(jnp.int32)

  pad_n = E_PAD - N_EDGES
  pad_ar = jnp.arange(pad_n, dtype=jnp.int32)
  pad_src = (pad_ar * 131) % N_NODES
  pad_dst = N_NODES + (pad_ar % 112)
  src_rows = jnp.concatenate([src, pad_src]).reshape(E_PAD // CHUNK, CHUNK)
  dst_rows = jnp.concatenate([dst, pad_dst]).reshape(E_PAD // CHUNK, CHUNK)

  zeros_r = jnp.zeros((N_PAD, D), jnp.float32)
  ones_c = jnp.ones((CHUNK, D), jnp.float32)

  b1r = jnp.broadcast_to(b1, (8, D))
  b2r = jnp.broadcast_to(b2, (8, D))

  c = _sc_count(dst_rows, zeros_r, ones_c)
  p = _sc_aggregate(x, src_rows, dst_rows, zeros_r)
  h = _tc_layer(p, c, x, W1_l, W1_r, b1r, relu=True)

  q = _sc_aggregate(h, src_rows, dst_rows, zeros_r)
  out = _tc_layer(q, c, h, W2_l, W2_r, b2r, relu=False)
  return out
